# baseline (speedup 1.0000x reference)
import functools

import jax
import jax.numpy as jnp
from jax import lax
from jax.experimental import pallas as pl
from jax.experimental.pallas import tpu as pltpu
from jax.experimental.pallas import tpu_sc as plsc

N = 10000
E = 320000
D_IN = 128
D_H = 256
D_OUT = 128

NCORE = 2
NSUB = 16
NW = NCORE * NSUB
EPW = E // NW
NB = 80
BB = EPW // NB
NBH = NB // 2
RPT = N // NSUB
BN = 1000

_sc_mesh = plsc.VectorSubcoreMesh(core_axis_name="c", subcore_axis_name="s")


ROWS_A = 624
ROWS_LAST = N - ROWS_A * (NSUB - 1)


def _sliced_rows(s, fn):
    @pl.when(s < NSUB - 1)
    def _():
        fn(pl.multiple_of(s * ROWS_A, 8), ROWS_A)

    @pl.when(s == NSUB - 1)
    def _():
        fn(ROWS_A * (NSUB - 1), ROWS_LAST)


@functools.partial(
    pl.kernel,
    mesh=_sc_mesh,
    out_type=jax.ShapeDtypeStruct((NCORE, N, 128), jnp.float32),
    scratch_types=[
        pltpu.VMEM((NBH, BB), jnp.int32),
        pltpu.VMEM((NBH, BB), jnp.int32),
        pltpu.VMEM((BB, 128), jnp.float32),
        pltpu.VMEM((BB, 128), jnp.float32),
        pltpu.VMEM_SHARED((N, 128), jnp.float32),
    ],
)
def _deg_kernel(src_hbm, dst_hbm, vconst_hbm, zmat_hbm, out_hbm,
                src_v, dst_v, vs, vd, hist):
    c = lax.axis_index("c")
    s = lax.axis_index("s")
    w = c * NSUB + s
    pltpu.sync_copy(vconst_hbm.at[0], vs)
    pltpu.sync_copy(vconst_hbm.at[1], vd)
    _sliced_rows(s, lambda o, r: pltpu.sync_copy(
        zmat_hbm.at[pl.ds(o, r)], hist.at[pl.ds(o, r)]))
    plsc.subcore_barrier()
    for h in range(2):
        pltpu.sync_copy(src_hbm.at[w, pl.ds(h * NBH, NBH)], src_v)
        pltpu.sync_copy(dst_hbm.at[w, pl.ds(h * NBH, NBH)], dst_v)

        def body(j, _):
            pltpu.sync_copy(vs, hist.at[src_v.at[j]], add=True)
            pltpu.sync_copy(vd, hist.at[dst_v.at[j]], add=True)
            return 0

        lax.fori_loop(0, NBH, body, 0)
    plsc.subcore_barrier()
    _sliced_rows(s, lambda o, r: pltpu.sync_copy(
        hist.at[pl.ds(o, r)], out_hbm.at[c, pl.ds(o, r)]))


@functools.partial(
    pl.kernel,
    mesh=_sc_mesh,
    out_type=jax.ShapeDtypeStruct((NCORE, N, 128), jnp.float32),
    scratch_types=[
        pltpu.VMEM((NBH, BB), jnp.int32),
        pltpu.VMEM((NBH, BB), jnp.int32),
        pltpu.VMEM((BB, 128), jnp.float32),
        pltpu.VMEM((BB, 128), jnp.float32),
        pltpu.VMEM_SHARED((N, 128), jnp.float32),
        pltpu.SemaphoreType.DMA,
        pltpu.SemaphoreType.DMA,
        pltpu.SemaphoreType.DMA,
        pltpu.SemaphoreType.DMA,
    ],
)
def _spmm_kernel(h_hbm, src_hbm, dst_hbm, zmat_hbm, out_hbm,
                 src_v, dst_v, gbuf0, gbuf1, agg_sh,
                 gsem0, gsem1, ssem0, ssem1):
    c = lax.axis_index("c")
    s = lax.axis_index("s")
    w = c * NSUB + s
    _sliced_rows(s, lambda o, r: pltpu.sync_copy(
        zmat_hbm.at[pl.ds(o, r)], agg_sh.at[pl.ds(o, r)]))
    plsc.subcore_barrier()

    for h in range(2):
        pltpu.sync_copy(src_hbm.at[w, pl.ds(h * NBH, NBH)], src_v)
        pltpu.sync_copy(dst_hbm.at[w, pl.ds(h * NBH, NBH)], dst_v)
        pltpu.async_copy(h_hbm.at[src_v.at[0]], gbuf0, gsem0)
        pltpu.async_copy(h_hbm.at[src_v.at[1]], gbuf1, gsem1)

        def body(jj, _):
            j = 2 * jj
            pltpu.make_async_copy(h_hbm.at[src_v.at[j]], gbuf0, gsem0).wait()
            pltpu.async_copy(gbuf0, agg_sh.at[dst_v.at[j]], ssem0, add=True)
            pltpu.make_async_copy(
                h_hbm.at[src_v.at[j + 1]], gbuf1, gsem1).wait()
            pltpu.async_copy(gbuf1, agg_sh.at[dst_v.at[j + 1]], ssem1,
                             add=True)

            @pl.when(jj < NBH // 2 - 1)
            def _():
                pltpu.make_async_copy(
                    gbuf0, agg_sh.at[dst_v.at[j]], ssem0).wait()
                pltpu.async_copy(h_hbm.at[src_v.at[j + 2]], gbuf0, gsem0)
                pltpu.make_async_copy(
                    gbuf1, agg_sh.at[dst_v.at[j + 1]], ssem1).wait()
                pltpu.async_copy(h_hbm.at[src_v.at[j + 3]], gbuf1, gsem1)

            return 0

        lax.fori_loop(0, NBH // 2, body, 0)
        pltpu.make_async_copy(
            gbuf0, agg_sh.at[dst_v.at[NBH - 2]], ssem0).wait()
        pltpu.make_async_copy(
            gbuf1, agg_sh.at[dst_v.at[NBH - 1]], ssem1).wait()
    plsc.subcore_barrier()
    _sliced_rows(s, lambda o, r: pltpu.sync_copy(
        agg_sh.at[pl.ds(o, r)], out_hbm.at[c, pl.ds(o, r)]))


def _prep_body(hist_ref, x_ref, xs_ref, ns_ref, nd_ref):
    h = hist_ref[...]
    deg_o = (h[0] + h[1])[:, 0:1]
    deg_i = (h[0] + h[1])[:, 8:9]
    ns = lax.rsqrt(jnp.maximum(deg_o, 1.0))
    nd = lax.rsqrt(jnp.maximum(deg_i, 1.0))
    xs_ref[...] = x_ref[...] * ns
    ns_ref[...] = ns
    nd_ref[...] = nd


def _prep_call(hist3, x):
    return pl.pallas_call(
        _prep_body,
        grid=(N // BN,),
        in_specs=[
            pl.BlockSpec((NCORE, BN, 128), lambda i: (0, i, 0)),
            pl.BlockSpec((BN, D_IN), lambda i: (i, 0)),
        ],
        out_specs=[
            pl.BlockSpec((BN, D_IN), lambda i: (i, 0)),
            pl.BlockSpec((BN, 1), lambda i: (i, 0)),
            pl.BlockSpec((BN, 1), lambda i: (i, 0)),
        ],
        out_shape=[
            jax.ShapeDtypeStruct((N, D_IN), jnp.float32),
            jax.ShapeDtypeStruct((N, 1), jnp.float32),
            jax.ShapeDtypeStruct((N, 1), jnp.float32),
        ],
    )(hist3, x)


def _mid_body(p_ref, w1_ref, b1_ref, ns_ref, nd_ref, w2_ref, h2_ref):
    p = p_ref[0] + p_ref[1]
    a = jnp.dot(p, w1_ref[...], preferred_element_type=jnp.float32)
    o1 = a * nd_ref[...] + b1_ref[...]
    r = jnp.maximum(o1, 0.0) * ns_ref[...]
    h2_ref[...] = jnp.dot(r, w2_ref[...], preferred_element_type=jnp.float32)


def _mid_call(p, W1, b1r, ns, nd, W2):
    return pl.pallas_call(
        _mid_body,
        grid=(N // BN,),
        in_specs=[
            pl.BlockSpec((NCORE, BN, 128), lambda i: (0, i, 0)),
            pl.BlockSpec((D_IN, D_H), lambda i: (0, 0)),
            pl.BlockSpec((1, D_H), lambda i: (0, 0)),
            pl.BlockSpec((BN, 1), lambda i: (i, 0)),
            pl.BlockSpec((BN, 1), lambda i: (i, 0)),
            pl.BlockSpec((D_H, D_OUT), lambda i: (0, 0)),
        ],
        out_specs=pl.BlockSpec((BN, D_OUT), lambda i: (i, 0)),
        out_shape=jax.ShapeDtypeStruct((N, D_OUT), jnp.float32),
    )(p, W1, b1r, ns, nd, W2)


def _fin_body(q_ref, nd_ref, b2_ref, o_ref):
    o_ref[...] = (q_ref[0] + q_ref[1]) * nd_ref[...] + b2_ref[...]


def _fin_call(q, nd, b2r):
    return pl.pallas_call(
        _fin_body,
        grid=(N // BN,),
        in_specs=[
            pl.BlockSpec((NCORE, BN, D_OUT), lambda i: (0, i, 0)),
            pl.BlockSpec((BN, 1), lambda i: (i, 0)),
            pl.BlockSpec((1, D_OUT), lambda i: (0, 0)),
        ],
        out_specs=pl.BlockSpec((BN, D_OUT), lambda i: (i, 0)),
        out_shape=jax.ShapeDtypeStruct((N, D_OUT), jnp.float32),
    )(q, nd, b2r)


def kernel(features, edge_index, W1, b1, W2, b2):
    ei = edge_index.astype(jnp.int32)
    src4 = ei[0].reshape(NW, NB, BB)
    dst4 = ei[1].reshape(NW, NB, BB)
    lane = lax.broadcasted_iota(jnp.int32, (2, BB, 128), 2)
    half = lax.broadcasted_iota(jnp.int32, (2, BB, 128), 0)
    vconst = jnp.where((lane // 8) == half, 1.0, 0.0).astype(jnp.float32)
    zmat = jnp.zeros((N, 128), jnp.float32)

    hist3 = _deg_kernel(src4, dst4, vconst, zmat)
    xs, ns, nd = _prep_call(hist3, features)
    p = _spmm_kernel(xs, src4, dst4, zmat)
    h2 = _mid_call(p, W1, b1.reshape(1, D_H), ns, nd, W2)
    q = _spmm_kernel(h2, src4, dst4, zmat)
    return _fin_call(q, nd, b2.reshape(1, D_OUT))


# speedup vs baseline: 1.0587x; 1.0587x over previous
import functools

import jax
import jax.numpy as jnp
from jax import lax
from jax.experimental import pallas as pl
from jax.experimental.pallas import tpu as pltpu
from jax.experimental.pallas import tpu_sc as plsc

N = 10000
E = 320000
D_IN = 128
D_H = 256
D_OUT = 128

NCORE = 2
NSUB = 16
NW = NCORE * NSUB
EPW = E // NW
NB = 80
BB = EPW // NB
NBH = NB // 2
RPT = N // NSUB
BN = 1000

_sc_mesh = plsc.VectorSubcoreMesh(core_axis_name="c", subcore_axis_name="s")


ROWS_A = 624
ROWS_LAST = N - ROWS_A * (NSUB - 1)


def _sliced_rows(s, fn):
    @pl.when(s < NSUB - 1)
    def _():
        fn(pl.multiple_of(s * ROWS_A, 8), ROWS_A)

    @pl.when(s == NSUB - 1)
    def _():
        fn(ROWS_A * (NSUB - 1), ROWS_LAST)


@functools.partial(
    pl.kernel,
    mesh=_sc_mesh,
    out_type=jax.ShapeDtypeStruct((NCORE, N, 128), jnp.float32),
    scratch_types=[
        pltpu.VMEM((NBH, BB), jnp.int32),
        pltpu.VMEM((NBH, BB), jnp.int32),
        pltpu.VMEM((BB, 128), jnp.float32),
        pltpu.VMEM((BB, 128), jnp.float32),
        pltpu.VMEM_SHARED((N, 128), jnp.float32),
    ],
)
def _deg_kernel(src_hbm, dst_hbm, vconst_hbm, zmat_hbm, out_hbm,
                src_v, dst_v, vs, vd, hist):
    c = lax.axis_index("c")
    s = lax.axis_index("s")
    w = c * NSUB + s
    pltpu.sync_copy(vconst_hbm.at[0], vs)
    pltpu.sync_copy(vconst_hbm.at[1], vd)
    _sliced_rows(s, lambda o, r: pltpu.sync_copy(
        zmat_hbm.at[pl.ds(o, r)], hist.at[pl.ds(o, r)]))
    plsc.subcore_barrier()
    for h in range(2):
        pltpu.sync_copy(src_hbm.at[w, pl.ds(h * NBH, NBH)], src_v)
        pltpu.sync_copy(dst_hbm.at[w, pl.ds(h * NBH, NBH)], dst_v)

        def body(j, _):
            pltpu.sync_copy(vs, hist.at[src_v.at[j]], add=True)
            pltpu.sync_copy(vd, hist.at[dst_v.at[j]], add=True)
            return 0

        lax.fori_loop(0, NBH, body, 0)
    plsc.subcore_barrier()
    _sliced_rows(s, lambda o, r: pltpu.sync_copy(
        hist.at[pl.ds(o, r)], out_hbm.at[c, pl.ds(o, r)]))


@functools.partial(
    pl.kernel,
    mesh=_sc_mesh,
    out_type=jax.ShapeDtypeStruct((NCORE, N, 128), jnp.float32),
    scratch_types=[
        pltpu.VMEM((NBH, BB), jnp.int32),
        pltpu.VMEM((NBH, BB), jnp.int32),
        pltpu.VMEM((BB, 128), jnp.float32),
        pltpu.VMEM((BB, 128), jnp.float32),
        pltpu.VMEM_SHARED((N, 128), jnp.float32),
        pltpu.SemaphoreType.DMA,
        pltpu.SemaphoreType.DMA,
    ],
)
def _spmm_kernel(h_hbm, src_hbm, dst_hbm, zmat_hbm, out_hbm,
                 src_v, dst_v, gbuf0, gbuf1, agg_sh, gsem0, gsem1):
    c = lax.axis_index("c")
    s = lax.axis_index("s")
    w = c * NSUB + s
    _sliced_rows(s, lambda o, r: pltpu.sync_copy(
        zmat_hbm.at[pl.ds(o, r)], agg_sh.at[pl.ds(o, r)]))
    plsc.subcore_barrier()

    for h in range(2):
        pltpu.sync_copy(src_hbm.at[w, pl.ds(h * NBH, NBH)], src_v)
        pltpu.sync_copy(dst_hbm.at[w, pl.ds(h * NBH, NBH)], dst_v)
        pltpu.async_copy(h_hbm.at[src_v.at[0]], gbuf0, gsem0)

        def body(jj, _):
            j = 2 * jj
            pltpu.make_async_copy(h_hbm.at[src_v.at[j]], gbuf0, gsem0).wait()
            pltpu.async_copy(h_hbm.at[src_v.at[j + 1]], gbuf1, gsem1)
            pltpu.sync_copy(gbuf0, agg_sh.at[dst_v.at[j]], add=True)
            pltpu.make_async_copy(
                h_hbm.at[src_v.at[j + 1]], gbuf1, gsem1).wait()

            @pl.when(jj < NBH // 2 - 1)
            def _():
                pltpu.async_copy(h_hbm.at[src_v.at[j + 2]], gbuf0, gsem0)

            pltpu.sync_copy(gbuf1, agg_sh.at[dst_v.at[j + 1]], add=True)
            return 0

        lax.fori_loop(0, NBH // 2, body, 0)
    plsc.subcore_barrier()
    _sliced_rows(s, lambda o, r: pltpu.sync_copy(
        agg_sh.at[pl.ds(o, r)], out_hbm.at[c, pl.ds(o, r)]))


def _prep_body(hist_ref, x_ref, xs_ref, ns_ref, nd_ref):
    h = hist_ref[...]
    deg_o = (h[0] + h[1])[:, 0:1]
    deg_i = (h[0] + h[1])[:, 8:9]
    ns = lax.rsqrt(jnp.maximum(deg_o, 1.0))
    nd = lax.rsqrt(jnp.maximum(deg_i, 1.0))
    xs_ref[...] = x_ref[...] * ns
    ns_ref[...] = ns
    nd_ref[...] = nd


def _prep_call(hist3, x):
    return pl.pallas_call(
        _prep_body,
        grid=(N // BN,),
        in_specs=[
            pl.BlockSpec((NCORE, BN, 128), lambda i: (0, i, 0)),
            pl.BlockSpec((BN, D_IN), lambda i: (i, 0)),
        ],
        out_specs=[
            pl.BlockSpec((BN, D_IN), lambda i: (i, 0)),
            pl.BlockSpec((BN, 1), lambda i: (i, 0)),
            pl.BlockSpec((BN, 1), lambda i: (i, 0)),
        ],
        out_shape=[
            jax.ShapeDtypeStruct((N, D_IN), jnp.float32),
            jax.ShapeDtypeStruct((N, 1), jnp.float32),
            jax.ShapeDtypeStruct((N, 1), jnp.float32),
        ],
    )(hist3, x)


def _mid_body(p_ref, w1_ref, b1_ref, ns_ref, nd_ref, w2_ref, h2_ref):
    p = p_ref[0] + p_ref[1]
    a = jnp.dot(p, w1_ref[...], preferred_element_type=jnp.float32)
    o1 = a * nd_ref[...] + b1_ref[...]
    r = jnp.maximum(o1, 0.0) * ns_ref[...]
    h2_ref[...] = jnp.dot(r, w2_ref[...], preferred_element_type=jnp.float32)


def _mid_call(p, W1, b1r, ns, nd, W2):
    return pl.pallas_call(
        _mid_body,
        grid=(N // BN,),
        in_specs=[
            pl.BlockSpec((NCORE, BN, 128), lambda i: (0, i, 0)),
            pl.BlockSpec((D_IN, D_H), lambda i: (0, 0)),
            pl.BlockSpec((1, D_H), lambda i: (0, 0)),
            pl.BlockSpec((BN, 1), lambda i: (i, 0)),
            pl.BlockSpec((BN, 1), lambda i: (i, 0)),
            pl.BlockSpec((D_H, D_OUT), lambda i: (0, 0)),
        ],
        out_specs=pl.BlockSpec((BN, D_OUT), lambda i: (i, 0)),
        out_shape=jax.ShapeDtypeStruct((N, D_OUT), jnp.float32),
    )(p, W1, b1r, ns, nd, W2)


def _fin_body(q_ref, nd_ref, b2_ref, o_ref):
    o_ref[...] = (q_ref[0] + q_ref[1]) * nd_ref[...] + b2_ref[...]


def _fin_call(q, nd, b2r):
    return pl.pallas_call(
        _fin_body,
        grid=(N // BN,),
        in_specs=[
            pl.BlockSpec((NCORE, BN, D_OUT), lambda i: (0, i, 0)),
            pl.BlockSpec((BN, 1), lambda i: (i, 0)),
            pl.BlockSpec((1, D_OUT), lambda i: (0, 0)),
        ],
        out_specs=pl.BlockSpec((BN, D_OUT), lambda i: (i, 0)),
        out_shape=jax.ShapeDtypeStruct((N, D_OUT), jnp.float32),
    )(q, nd, b2r)


def kernel(features, edge_index, W1, b1, W2, b2):
    ei = edge_index.astype(jnp.int32)
    src4 = ei[0].reshape(NW, NB, BB)
    dst4 = ei[1].reshape(NW, NB, BB)
    lane = lax.broadcasted_iota(jnp.int32, (2, BB, 128), 2)
    half = lax.broadcasted_iota(jnp.int32, (2, BB, 128), 0)
    vconst = jnp.where((lane // 8) == half, 1.0, 0.0).astype(jnp.float32)
    zmat = jnp.zeros((N, 128), jnp.float32)

    hist3 = _deg_kernel(src4, dst4, vconst, zmat)
    xs, ns, nd = _prep_call(hist3, features)
    p = _spmm_kernel(xs, src4, dst4, zmat)
    h2 = _mid_call(p, W1, b1.reshape(1, D_H), ns, nd, W2)
    q = _spmm_kernel(h2, src4, dst4, zmat)
    return _fin_call(q, nd, b2.reshape(1, D_OUT))


# speedup vs baseline: 1.0768x; 1.0171x over previous
import functools

import jax
import jax.numpy as jnp
from jax import lax
from jax.experimental import pallas as pl
from jax.experimental.pallas import tpu as pltpu
from jax.experimental.pallas import tpu_sc as plsc

N = 10000
E = 320000
D_IN = 128
D_H = 256
D_OUT = 128

NCORE = 2
NSUB = 16
NW = NCORE * NSUB
EPW = E // NW
NB = 80
BB = EPW // NB
NBH = NB // 2
RPT = N // NSUB
BN = 2000
ZCH = 120

_sc_mesh = plsc.VectorSubcoreMesh(core_axis_name="c", subcore_axis_name="s")


ROWS_A = 624
ROWS_LAST = N - ROWS_A * (NSUB - 1)


def _sliced_rows(s, fn):
    @pl.when(s < NSUB - 1)
    def _():
        fn(pl.multiple_of(s * ROWS_A, 8), ROWS_A)

    @pl.when(s == NSUB - 1)
    def _():
        fn(ROWS_A * (NSUB - 1), ROWS_LAST)


def _zero_rows(zbuf, dst, o, r):
    for k in range(0, r, ZCH):
        n = min(ZCH, r - k)
        pltpu.sync_copy(zbuf.at[pl.ds(0, n)], dst.at[pl.ds(o + k, n)])


@functools.partial(
    pl.kernel,
    mesh=_sc_mesh,
    out_type=jax.ShapeDtypeStruct((NCORE, N, 128), jnp.float32),
    scratch_types=[
        pltpu.VMEM((NBH, BB), jnp.int32),
        pltpu.VMEM((NBH, BB), jnp.int32),
        pltpu.VMEM((BB, 128), jnp.float32),
        pltpu.VMEM((BB, 128), jnp.float32),
        pltpu.VMEM_SHARED((N, 128), jnp.float32),
    ],
)
def _deg_kernel(src_hbm, dst_hbm, vconst_hbm, zmat_hbm, out_hbm,
                src_v, dst_v, vs, vd, hist):
    c = lax.axis_index("c")
    s = lax.axis_index("s")
    w = c * NSUB + s
    pltpu.sync_copy(zmat_hbm, vs)
    _sliced_rows(s, lambda o, r: _zero_rows(vs, hist, o, r))
    pltpu.sync_copy(vconst_hbm.at[0], vs)
    pltpu.sync_copy(vconst_hbm.at[1], vd)
    plsc.subcore_barrier()
    for h in range(2):
        pltpu.sync_copy(src_hbm.at[w, pl.ds(h * NBH, NBH)], src_v)
        pltpu.sync_copy(dst_hbm.at[w, pl.ds(h * NBH, NBH)], dst_v)

        def body(j, _):
            pltpu.sync_copy(vs, hist.at[src_v.at[j]], add=True)
            pltpu.sync_copy(vd, hist.at[dst_v.at[j]], add=True)
            return 0

        lax.fori_loop(0, NBH, body, 0)
    plsc.subcore_barrier()
    _sliced_rows(s, lambda o, r: pltpu.sync_copy(
        hist.at[pl.ds(o, r)], out_hbm.at[c, pl.ds(o, r)]))


@functools.partial(
    pl.kernel,
    mesh=_sc_mesh,
    out_type=jax.ShapeDtypeStruct((NCORE, N, 128), jnp.float32),
    scratch_types=[
        pltpu.VMEM((NBH, BB), jnp.int32),
        pltpu.VMEM((NBH, BB), jnp.int32),
        pltpu.VMEM((BB, 128), jnp.float32),
        pltpu.VMEM((BB, 128), jnp.float32),
        pltpu.VMEM_SHARED((N, 128), jnp.float32),
        pltpu.SemaphoreType.DMA,
        pltpu.SemaphoreType.DMA,
    ],
)
def _spmm_kernel(h_hbm, src_hbm, dst_hbm, zmat_hbm, out_hbm,
                 src_v, dst_v, gbuf0, gbuf1, agg_sh, gsem0, gsem1):
    c = lax.axis_index("c")
    s = lax.axis_index("s")
    w = c * NSUB + s
    pltpu.sync_copy(zmat_hbm, gbuf0)
    _sliced_rows(s, lambda o, r: _zero_rows(gbuf0, agg_sh, o, r))
    plsc.subcore_barrier()

    for h in range(2):
        pltpu.sync_copy(src_hbm.at[w, pl.ds(h * NBH, NBH)], src_v)
        pltpu.sync_copy(dst_hbm.at[w, pl.ds(h * NBH, NBH)], dst_v)
        pltpu.async_copy(h_hbm.at[src_v.at[0]], gbuf0, gsem0)

        def body(jj, _):
            j = 2 * jj
            pltpu.make_async_copy(h_hbm.at[src_v.at[j]], gbuf0, gsem0).wait()
            pltpu.async_copy(h_hbm.at[src_v.at[j + 1]], gbuf1, gsem1)
            pltpu.sync_copy(gbuf0, agg_sh.at[dst_v.at[j]], add=True)
            pltpu.make_async_copy(
                h_hbm.at[src_v.at[j + 1]], gbuf1, gsem1).wait()

            @pl.when(jj < NBH // 2 - 1)
            def _():
                pltpu.async_copy(h_hbm.at[src_v.at[j + 2]], gbuf0, gsem0)

            pltpu.sync_copy(gbuf1, agg_sh.at[dst_v.at[j + 1]], add=True)
            return 0

        lax.fori_loop(0, NBH // 2, body, 0)
    plsc.subcore_barrier()
    _sliced_rows(s, lambda o, r: pltpu.sync_copy(
        agg_sh.at[pl.ds(o, r)], out_hbm.at[c, pl.ds(o, r)]))


def _prep_body(hist_ref, x_ref, xs_ref, ns_ref, nd_ref):
    h = hist_ref[...]
    deg_o = (h[0] + h[1])[:, 0:1]
    deg_i = (h[0] + h[1])[:, 8:9]
    ns = lax.rsqrt(jnp.maximum(deg_o, 1.0))
    nd = lax.rsqrt(jnp.maximum(deg_i, 1.0))
    xs_ref[...] = x_ref[...] * ns
    ns_ref[...] = ns
    nd_ref[...] = nd


def _prep_call(hist3, x):
    return pl.pallas_call(
        _prep_body,
        grid=(N // BN,),
        in_specs=[
            pl.BlockSpec((NCORE, BN, 128), lambda i: (0, i, 0)),
            pl.BlockSpec((BN, D_IN), lambda i: (i, 0)),
        ],
        out_specs=[
            pl.BlockSpec((BN, D_IN), lambda i: (i, 0)),
            pl.BlockSpec((BN, 1), lambda i: (i, 0)),
            pl.BlockSpec((BN, 1), lambda i: (i, 0)),
        ],
        out_shape=[
            jax.ShapeDtypeStruct((N, D_IN), jnp.float32),
            jax.ShapeDtypeStruct((N, 1), jnp.float32),
            jax.ShapeDtypeStruct((N, 1), jnp.float32),
        ],
    )(hist3, x)


def _mid_body(p_ref, w1_ref, b1_ref, ns_ref, nd_ref, w2_ref, h2_ref):
    p = p_ref[0] + p_ref[1]
    a = jnp.dot(p, w1_ref[...], preferred_element_type=jnp.float32)
    o1 = a * nd_ref[...] + b1_ref[...]
    r = jnp.maximum(o1, 0.0) * ns_ref[...]
    h2_ref[...] = jnp.dot(r, w2_ref[...], preferred_element_type=jnp.float32)


def _mid_call(p, W1, b1r, ns, nd, W2):
    return pl.pallas_call(
        _mid_body,
        grid=(N // BN,),
        in_specs=[
            pl.BlockSpec((NCORE, BN, 128), lambda i: (0, i, 0)),
            pl.BlockSpec((D_IN, D_H), lambda i: (0, 0)),
            pl.BlockSpec((1, D_H), lambda i: (0, 0)),
            pl.BlockSpec((BN, 1), lambda i: (i, 0)),
            pl.BlockSpec((BN, 1), lambda i: (i, 0)),
            pl.BlockSpec((D_H, D_OUT), lambda i: (0, 0)),
        ],
        out_specs=pl.BlockSpec((BN, D_OUT), lambda i: (i, 0)),
        out_shape=jax.ShapeDtypeStruct((N, D_OUT), jnp.float32),
    )(p, W1, b1r, ns, nd, W2)


def _fin_body(q_ref, nd_ref, b2_ref, o_ref):
    o_ref[...] = (q_ref[0] + q_ref[1]) * nd_ref[...] + b2_ref[...]


def _fin_call(q, nd, b2r):
    return pl.pallas_call(
        _fin_body,
        grid=(N // BN,),
        in_specs=[
            pl.BlockSpec((NCORE, BN, D_OUT), lambda i: (0, i, 0)),
            pl.BlockSpec((BN, 1), lambda i: (i, 0)),
            pl.BlockSpec((1, D_OUT), lambda i: (0, 0)),
        ],
        out_specs=pl.BlockSpec((BN, D_OUT), lambda i: (i, 0)),
        out_shape=jax.ShapeDtypeStruct((N, D_OUT), jnp.float32),
    )(q, nd, b2r)


def kernel(features, edge_index, W1, b1, W2, b2):
    ei = edge_index.astype(jnp.int32)
    src4 = ei[0].reshape(NW, NB, BB)
    dst4 = ei[1].reshape(NW, NB, BB)
    lane = lax.broadcasted_iota(jnp.int32, (2, BB, 128), 2)
    half = lax.broadcasted_iota(jnp.int32, (2, BB, 128), 0)
    vconst = jnp.where((lane // 8) == half, 1.0, 0.0).astype(jnp.float32)
    zmat = jnp.zeros((BB, 128), jnp.float32)

    hist3 = _deg_kernel(src4, dst4, vconst, zmat)
    xs, ns, nd = _prep_call(hist3, features)
    p = _spmm_kernel(xs, src4, dst4, zmat)
    h2 = _mid_call(p, W1, b1.reshape(1, D_H), ns, nd, W2)
    q = _spmm_kernel(h2, src4, dst4, zmat)
    return _fin_call(q, nd, b2.reshape(1, D_OUT))
